# baseline (speedup 1.0000x reference)
import math

import jax
import jax.numpy as jnp
from jax.experimental import pallas as pl
from jax.experimental.pallas import tpu as pltpu

N_CLS = 32000
PAD = 0
EPS = 0.1 / (N_CLS - 2)
CONF = 0.9
C0 = (N_CLS - 2) * EPS * math.log(EPS) + CONF * math.log(CONF)

RBLK = 128


def _body(tgt_ref, x_ref, out_ref):
    j = pl.program_id(0)
    x = x_ref[...]
    tgt = tgt_ref[...]
    tmask = tgt != PAD
    mf = tmask.astype(jnp.float32)

    colsum = jax.lax.dot_general(mf, x, (((0,), (0,)), ((), ())),
                                 preferred_element_type=jnp.float32)
    row_total = jnp.sum(colsum)
    col0_masked = colsum[0, 0]

    col = jax.lax.broadcasted_iota(jnp.int32, (1, N_CLS), 1)
    part2_raw = jnp.sum(jnp.where(col == tgt, x, 0.0))
    col0_all = jnp.sum(x[:, 0:1])
    part2 = part2_raw - (col0_all - col0_masked)

    cnt = jnp.sum(mf)
    part = (C0 * cnt - EPS * (row_total - col0_masked)
            - (CONF - EPS) * part2)

    @pl.when(j == 0)
    def _init():
        out_ref[0, 0] = part

    @pl.when(j != 0)
    def _acc():
        out_ref[0, 0] += part


def kernel(x, target):
    n, c = x.shape
    out = pl.pallas_call(
        _body,
        grid=(n // RBLK,),
        in_specs=[
            pl.BlockSpec((RBLK, 1), lambda j: (j, 0)),
            pl.BlockSpec((RBLK, c), lambda j: (j, 0)),
        ],
        out_specs=pl.BlockSpec((1, 1), lambda j: (0, 0),
                               memory_space=pltpu.SMEM),
        out_shape=jax.ShapeDtypeStruct((1, 1), jnp.float32),
    )(target.reshape(n, 1), x)
    return out[0, 0]


# speedup vs baseline: 1.1466x; 1.1466x over previous
import math

import jax
import jax.numpy as jnp
from jax.experimental import pallas as pl
from jax.experimental.pallas import tpu as pltpu

N_CLS = 32000
PAD = 0
EPS = 0.1 / (N_CLS - 2)
CONF = 0.9
C0 = (N_CLS - 2) * EPS * math.log(EPS) + CONF * math.log(CONF)

RBLK = 128


def _body(tgt_ref, x_ref, out_ref):
    j = pl.program_id(0)
    x = x_ref[...]
    tgt = tgt_ref[...]
    tmask = tgt != PAD

    rs = jnp.sum(x, axis=1, keepdims=True)
    part_masked = jnp.sum(jnp.where(tmask, rs, 0.0))
    col = jax.lax.broadcasted_iota(jnp.int32, (1, N_CLS), 1)
    pm = jnp.sum(jnp.where(col == tgt, x, 0.0), axis=1, keepdims=True)
    part_match = jnp.sum(pm)

    col0 = x[:, 0:1]
    col0_all = jnp.sum(col0)
    col0_masked = jnp.sum(jnp.where(tmask, col0, 0.0))
    cnt = jnp.sum(tmask.astype(jnp.float32))

    part = (C0 * cnt
            - EPS * (part_masked - col0_masked)
            - (CONF - EPS) * (part_match - (col0_all - col0_masked)))

    @pl.when(j == 0)
    def _init():
        out_ref[0, 0] = part

    @pl.when(j != 0)
    def _acc():
        out_ref[0, 0] += part


def kernel(x, target):
    n, c = x.shape
    out = pl.pallas_call(
        _body,
        grid=(n // RBLK,),
        in_specs=[
            pl.BlockSpec((RBLK, 1), lambda j: (j, 0)),
            pl.BlockSpec((RBLK, c), lambda j: (j, 0)),
        ],
        out_specs=pl.BlockSpec((1, 1), lambda j: (0, 0),
                               memory_space=pltpu.SMEM),
        out_shape=jax.ShapeDtypeStruct((1, 1), jnp.float32),
    )(target.reshape(n, 1), x)
    return out[0, 0]
